# baseline (speedup 1.0000x reference)
import functools

import jax
import jax.numpy as jnp
from jax.experimental import pallas as pl
from jax.experimental.pallas import tpu as pltpu

_K_TOP = 16
_NEG = -3.0e38


def _mask_kernel(q_ref, k_ref, out_ref, *, k_top):
    q = q_ref[0]
    k = k_ref[0]
    qn = q / jnp.maximum(jnp.sqrt(jnp.sum(q * q, axis=-1, keepdims=True)), 1e-12)
    kn = k / jnp.maximum(jnp.sqrt(jnp.sum(k * k, axis=-1, keepdims=True)), 1e-12)
    scores = jax.lax.dot_general(
        qn, kn, (((1,), (1,)), ((), ())),
        preferred_element_type=jnp.float32,
        precision=jax.lax.Precision.DEFAULT,
    )

    s = scores
    for _ in range(k_top - 1):
        m = jnp.max(s, axis=-1, keepdims=True)
        s = jnp.where(s >= m, _NEG, s)
    thresh = jnp.max(s, axis=-1, keepdims=True)
    out_ref[0] = (scores >= thresh).astype(jnp.float32)


def kernel(q_blocks, k_blocks):
    B, Qb, C = q_blocks.shape
    _, Bb, _ = k_blocks.shape
    k_top = min(_K_TOP, Bb)
    tq = min(256, Qb)
    grid = (B, Qb // tq)
    return pl.pallas_call(
        functools.partial(_mask_kernel, k_top=k_top),
        grid=grid,
        in_specs=[
            pl.BlockSpec((1, tq, C), lambda b, qt: (b, qt, 0)),
            pl.BlockSpec((1, Bb, C), lambda b, qt: (b, 0, 0)),
        ],
        out_specs=pl.BlockSpec((1, tq, Bb), lambda b, qt: (b, qt, 0)),
        out_shape=jax.ShapeDtypeStruct((B, Qb, Bb), jnp.float32),
        compiler_params=pltpu.CompilerParams(
            dimension_semantics=("parallel", "parallel"),
        ),
    )(q_blocks, k_blocks)


# speedup vs baseline: 1.0568x; 1.0568x over previous
import functools

import jax
import jax.numpy as jnp
from jax.experimental import pallas as pl
from jax.experimental.pallas import tpu as pltpu

_K_TOP = 16
_NEG = -3.0e38


def _mask_kernel(q_ref, k_ref, out_ref, *, k_top):
    q = q_ref[0]
    k = k_ref[0]
    qn = q / jnp.maximum(jnp.sqrt(jnp.sum(q * q, axis=-1, keepdims=True)), 1e-12)
    kn = k / jnp.maximum(jnp.sqrt(jnp.sum(k * k, axis=-1, keepdims=True)), 1e-12)
    scores = jax.lax.dot_general(
        qn, kn, (((1,), (1,)), ((), ())),
        preferred_element_type=jnp.float32,
        precision=jax.lax.Precision.DEFAULT,
    )

    m = jnp.max(scores, axis=-1, keepdims=True)
    for _ in range(k_top - 1):
        m = jnp.max(jnp.where(scores < m, scores, _NEG), axis=-1, keepdims=True)
    out_ref[0] = (scores >= m).astype(jnp.float32)


def kernel(q_blocks, k_blocks):
    B, Qb, C = q_blocks.shape
    _, Bb, _ = k_blocks.shape
    k_top = min(_K_TOP, Bb)
    tq = min(256, Qb)
    grid = (B, Qb // tq)
    return pl.pallas_call(
        functools.partial(_mask_kernel, k_top=k_top),
        grid=grid,
        in_specs=[
            pl.BlockSpec((1, tq, C), lambda b, qt: (b, qt, 0)),
            pl.BlockSpec((1, Bb, C), lambda b, qt: (b, 0, 0)),
        ],
        out_specs=pl.BlockSpec((1, tq, Bb), lambda b, qt: (b, qt, 0)),
        out_shape=jax.ShapeDtypeStruct((B, Qb, Bb), jnp.float32),
        compiler_params=pltpu.CompilerParams(
            dimension_semantics=("parallel", "parallel"),
        ),
    )(q_blocks, k_blocks)


# speedup vs baseline: 1.1429x; 1.0815x over previous
import functools

import jax
import jax.numpy as jnp
from jax.experimental import pallas as pl
from jax.experimental.pallas import tpu as pltpu

_K_TOP = 16
_NEG = -3.0e38


def _mask_kernel(q_ref, k_ref, out_ref, *, k_top):
    q = q_ref[0]
    k = k_ref[0]
    qn = q / jnp.maximum(jnp.sqrt(jnp.sum(q * q, axis=-1, keepdims=True)), 1e-12)
    kn = k / jnp.maximum(jnp.sqrt(jnp.sum(k * k, axis=-1, keepdims=True)), 1e-12)
    scores = jax.lax.dot_general(
        qn, kn, (((1,), (1,)), ((), ())),
        preferred_element_type=jnp.float32,
        precision=jax.lax.Precision.DEFAULT,
    )

    m = jnp.max(scores, axis=-1, keepdims=True)
    for _ in range(k_top - 1):
        m = jnp.max(jnp.where(scores < m, scores, _NEG), axis=-1, keepdims=True)
    out_ref[0] = (scores >= m).astype(jnp.float32)


def kernel(q_blocks, k_blocks):
    B, Qb, C = q_blocks.shape
    _, Bb, _ = k_blocks.shape
    k_top = min(_K_TOP, Bb)
    tq = min(512, Qb)
    grid = (B, Qb // tq)
    return pl.pallas_call(
        functools.partial(_mask_kernel, k_top=k_top),
        grid=grid,
        in_specs=[
            pl.BlockSpec((1, tq, C), lambda b, qt: (b, qt, 0)),
            pl.BlockSpec((1, Bb, C), lambda b, qt: (b, 0, 0)),
        ],
        out_specs=pl.BlockSpec((1, tq, Bb), lambda b, qt: (b, qt, 0)),
        out_shape=jax.ShapeDtypeStruct((B, Qb, Bb), jnp.float32),
        compiler_params=pltpu.CompilerParams(
            dimension_semantics=("parallel", "parallel"),
        ),
    )(q_blocks, k_blocks)
